# Initial kernel scaffold; baseline (speedup 1.0000x reference)
import jax
import jax.numpy as jnp
from jax.experimental import pallas as pl


def kernel(node_feat, edge_index):
    raise NotImplementedError("write your pallas kernel here")


# speedup vs baseline: 3.1198x; 3.1198x over previous
import functools

import jax
import jax.numpy as jnp
from jax import lax
from jax.experimental import pallas as pl
from jax.experimental.pallas import tpu as pltpu
from jax.experimental.pallas import tpu_sc as plsc

NC = 2
NS = 16
CPT = 4
ECHUNK = 2000


def _sc_edge_kernel(N, D, E):
    n_tiles = NC * NS
    n_chunks = E // ECHUNK
    groups = ECHUNK // 16
    e_per_tile = E // n_tiles
    hist_chunks = e_per_tile // ECHUNK

    mesh = plsc.VectorSubcoreMesh(core_axis_name="c", subcore_axis_name="s")

    @functools.partial(
        pl.kernel,
        out_type=[
            jax.ShapeDtypeStruct((n_tiles, CPT, N), jnp.float32),
            jax.ShapeDtypeStruct((n_tiles, N), jnp.float32),
        ],
        mesh=mesh,
        scratch_types=[
            pltpu.VMEM((CPT, N), jnp.float32),
            pltpu.VMEM((CPT, N), jnp.float32),
            pltpu.VMEM((N,), jnp.float32),
            pltpu.VMEM((ECHUNK,), jnp.int32),
            pltpu.VMEM((ECHUNK,), jnp.int32),
        ],
        compiler_params=pltpu.CompilerParams(needs_layout_passes=False),
    )
    def sc_kernel(nft_hbm, src_hbm, dst_hbm, z4_hbm,
                  acc_out, hist_out,
                  nf_v, acc_v, hist_v, src_c, dst_c):
        c = lax.axis_index("c")
        s = lax.axis_index("s")
        w = c * NS + s

        jvs = [jnp.full((16,), j, jnp.int32) for j in range(CPT)]
        ones16 = jnp.ones((16,), jnp.float32)

        pltpu.sync_copy(nft_hbm.at[w], nf_v)
        pltpu.sync_copy(z4_hbm, acc_v)
        pltpu.sync_copy(z4_hbm.at[0], hist_v)

        def chunk_body(ch, carry):
            base = ch * ECHUNK
            pltpu.sync_copy(src_hbm.at[pl.ds(base, ECHUNK)], src_c)
            pltpu.sync_copy(dst_hbm.at[pl.ds(base, ECHUNK)], dst_c)

            def group_body(k, kcarry):
                src16 = src_c[pl.ds(k * 16, 16)]
                dst16 = dst_c[pl.ds(k * 16, 16)]
                for j in range(CPT):
                    v = plsc.load_gather(nf_v, [jvs[j], src16])
                    plsc.addupdate_scatter(acc_v, [jvs[j], dst16], v)
                return kcarry

            return lax.fori_loop(0, groups, group_body, carry)

        lax.fori_loop(0, n_chunks, chunk_body, 0)

        def hist_chunk(ch, carry):
            base = w * e_per_tile + ch * ECHUNK
            pltpu.sync_copy(dst_hbm.at[pl.ds(base, ECHUNK)], dst_c)

            def group_body(k, kcarry):
                dst16 = dst_c[pl.ds(k * 16, 16)]
                plsc.addupdate_scatter(hist_v, [dst16], ones16)
                return kcarry

            return lax.fori_loop(0, groups, group_body, carry)

        lax.fori_loop(0, hist_chunks, hist_chunk, 0)

        pltpu.sync_copy(acc_v, acc_out.at[w])
        pltpu.sync_copy(hist_v, hist_out.at[w])

    return sc_kernel


def _combine_kernel(N, D, n_tiles):
    def body(acc_ref, hist_ref, nft_ref, out_ref):
        deg = jnp.sum(hist_ref[...], axis=0)
        t = acc_ref[...] + deg[None, :] * nft_ref[...]
        out_ref[...] = t.T

    return pl.pallas_call(
        body,
        out_shape=jax.ShapeDtypeStruct((N, D), jnp.float32),
    )


def kernel(node_feat, edge_index):
    N, D = node_feat.shape
    E = edge_index.shape[1]
    n_tiles = NC * NS
    src = edge_index[0]
    dst = edge_index[1]
    nft = node_feat.T.reshape(n_tiles, CPT, N)
    z4 = jnp.zeros((CPT, N), jnp.float32)
    acc, hist = _sc_edge_kernel(N, D, E)(nft, src, dst, z4)
    acc128 = acc.reshape(D, N)
    nft128 = nft.reshape(D, N)
    return _combine_kernel(N, D, n_tiles)(acc128, hist, nft128)


# speedup vs baseline: 9.9689x; 3.1954x over previous
import functools

import jax
import jax.numpy as jnp
from jax import lax
from jax.experimental import pallas as pl
from jax.experimental.pallas import tpu as pltpu
from jax.experimental.pallas import tpu_sc as plsc

NC = 2
NS = 16
CPT = 4
ECHUNK = 2000
NBUF = 2


def _sc_edge_kernel(N, D, E):
    n_tiles = NC * NS
    n_chunks = E // ECHUNK
    groups = ECHUNK // 16
    e_per_tile = E // n_tiles
    hist_chunks = e_per_tile // ECHUNK

    mesh = plsc.VectorSubcoreMesh(core_axis_name="c", subcore_axis_name="s")

    @functools.partial(
        pl.kernel,
        out_type=[
            jax.ShapeDtypeStruct((n_tiles, CPT, N), jnp.float32),
            jax.ShapeDtypeStruct((n_tiles, N), jnp.float32),
        ],
        mesh=mesh,
        scratch_types=[
            pltpu.VMEM((CPT, N), jnp.float32),
            pltpu.VMEM((CPT, N), jnp.float32),
            pltpu.VMEM((N,), jnp.float32),
            [pltpu.VMEM((ECHUNK,), jnp.int32) for _ in range(NBUF)],
            [pltpu.VMEM((ECHUNK,), jnp.int32) for _ in range(NBUF)],
            [pltpu.SemaphoreType.DMA for _ in range(NBUF)],
            [pltpu.SemaphoreType.DMA for _ in range(NBUF)],
        ],
        compiler_params=pltpu.CompilerParams(needs_layout_passes=False),
    )
    def sc_kernel(nft_hbm, src_hbm, dst_hbm, z4_hbm,
                  acc_out, hist_out,
                  nf_v, acc_v, hist_v, src_bufs, dst_bufs,
                  src_sems, dst_sems):
        c = lax.axis_index("c")
        s = lax.axis_index("s")
        w = c * NS + s

        jvs = [jnp.full((16,), j, jnp.int32) for j in range(CPT)]
        ones16 = jnp.ones((16,), jnp.float32)

        pltpu.sync_copy(nft_hbm.at[w], nf_v)
        pltpu.sync_copy(z4_hbm, acc_v)
        pltpu.sync_copy(z4_hbm.at[0], hist_v)

        def issue(ch, b):
            base = ch * ECHUNK
            pltpu.async_copy(src_hbm.at[pl.ds(base, ECHUNK)],
                             src_bufs[b], src_sems[b])
            pltpu.async_copy(dst_hbm.at[pl.ds(base, ECHUNK)],
                             dst_bufs[b], dst_sems[b])

        def wait(ch, b):
            base = ch * ECHUNK
            pltpu.make_async_copy(src_hbm.at[pl.ds(base, ECHUNK)],
                                  src_bufs[b], src_sems[b]).wait()
            pltpu.make_async_copy(dst_hbm.at[pl.ds(base, ECHUNK)],
                                  dst_bufs[b], dst_sems[b]).wait()

        def process(b):
            src_c = src_bufs[b]
            dst_c = dst_bufs[b]

            @plsc.parallel_loop(0, groups, unroll=5)
            def _(k):
                src16 = src_c[pl.ds(k * 16, 16)]
                dst16 = dst_c[pl.ds(k * 16, 16)]
                for j in range(CPT):
                    v = plsc.load_gather(nf_v, [jvs[j], src16])
                    plsc.addupdate_scatter(acc_v, [jvs[j], dst16], v)

        for b in range(NBUF):
            issue(b, b)

        def outer(i, carry):
            for b in range(NBUF):
                ch = i * NBUF + b
                wait(ch, b)
                process(b)

                @pl.when(ch + NBUF < n_chunks)
                def _():
                    issue(ch + NBUF, b)

            return carry

        lax.fori_loop(0, n_chunks // NBUF, outer, 0)

        def hist_chunk(ch, carry):
            base = w * e_per_tile + ch * ECHUNK
            pltpu.sync_copy(dst_hbm.at[pl.ds(base, ECHUNK)], dst_bufs[0])

            @plsc.parallel_loop(0, groups, unroll=5)
            def _(k):
                dst16 = dst_bufs[0][pl.ds(k * 16, 16)]
                plsc.addupdate_scatter(hist_v, [dst16], ones16)

            return carry

        lax.fori_loop(0, hist_chunks, hist_chunk, 0)

        pltpu.sync_copy(acc_v, acc_out.at[w])
        pltpu.sync_copy(hist_v, hist_out.at[w])

    return sc_kernel


def _combine_kernel(N, D, n_tiles):
    def body(acc_ref, hist_ref, nft_ref, out_ref):
        deg = jnp.sum(hist_ref[...], axis=0)
        t = acc_ref[...] + deg[None, :] * nft_ref[...]
        out_ref[...] = t.T

    return pl.pallas_call(
        body,
        out_shape=jax.ShapeDtypeStruct((N, D), jnp.float32),
    )


def kernel(node_feat, edge_index):
    N, D = node_feat.shape
    E = edge_index.shape[1]
    n_tiles = NC * NS
    src = edge_index[0]
    dst = edge_index[1]
    nft = node_feat.T.reshape(n_tiles, CPT, N)
    z4 = jnp.zeros((CPT, N), jnp.float32)
    acc, hist = _sc_edge_kernel(N, D, E)(nft, src, dst, z4)
    acc128 = acc.reshape(D, N)
    nft128 = nft.reshape(D, N)
    return _combine_kernel(N, D, n_tiles)(acc128, hist, nft128)
